# baseline (speedup 1.0000x reference)
import dataclasses
import functools

import jax
import jax.numpy as jnp
from jax import lax
from jax.experimental import pallas as pl
from jax.experimental.pallas import tpu as pltpu
from jax.experimental.pallas import tpu_sc as plsc

N_REAL = 10000
N_PAD = 10240
D = 128
NC, NS = 2, 16
NW = NC * NS
EB = 128
E_REAL = 320000
NH = 2
CSLOW = 1
NBH_SLOW = 18
NBH_FAST = 62
NB_SLOW = NH * NBH_SLOW
NB_FAST = NH * NBH_FAST
N_CHUNKS = NS * (NB_SLOW + NB_FAST)
E_PAD = N_CHUNKS * EB
EPW = E_PAD // NW
ROWS_PER_TILE = N_PAD // NS

_mesh = plsc.VectorSubcoreMesh(core_axis_name="c", subcore_axis_name="s")

_sc_params = pltpu.CompilerParams()
if "needs_layout_passes" in pltpu.CompilerParams.__dataclass_fields__:
    _sc_params = dataclasses.replace(_sc_params, needs_layout_passes=False)


@functools.partial(
    pl.kernel,
    mesh=_mesh,
    out_type=jax.ShapeDtypeStruct((NW, N_PAD), jnp.float32),
    compiler_params=_sc_params,
    scratch_types=[
        pltpu.VMEM((EPW,), jnp.int32),
        pltpu.VMEM((N_PAD,), jnp.float32),
        pltpu.SemaphoreType.DMA,
    ],
)
def _deg_kernel(dst_hbm, out_hbm, idx_v, hist_v, sem):
    c = lax.axis_index("c")
    s = lax.axis_index("s")
    w = c * NS + s
    cp = pltpu.async_copy(dst_hbm.at[w], idx_v, sem)
    z16 = jnp.zeros((16,), jnp.float32)

    @pl.loop(0, N_PAD // 16)
    def _(i):
        hist_v[pl.ds(i * 16, 16)] = z16

    cp.wait()
    ones16 = jnp.ones((16,), jnp.float32)

    @pl.loop(0, EPW // 16)
    def _(t):
        idx = idx_v[pl.ds(t * 16, 16)]
        plsc.addupdate_scatter(hist_v, [idx], ones16)

    pltpu.sync_copy(hist_v, out_hbm.at[w])


@functools.partial(
    pl.kernel,
    mesh=_mesh,
    out_type=jax.ShapeDtypeStruct((NC, N_PAD, D), jnp.float32),
    scratch_types=[
        pltpu.VMEM((NBH_FAST, EB), jnp.int32),
        pltpu.VMEM((NBH_FAST, EB), jnp.int32),
        pltpu.VMEM((EB, D), jnp.float32),
        pltpu.VMEM((EB, D), jnp.float32),
        pltpu.VMEM_SHARED((N_PAD, D), jnp.float32),
        pltpu.SemaphoreType.DMA,
        pltpu.SemaphoreType.DMA,
        pltpu.SemaphoreType.DMA,
    ],
)
def _scatter_kernel(g_hbm, src_hbm, dst_hbm, zeros_hbm, out_hbm,
                    sidx, didx, rows_a, rows_b, acc, sem, sem_a, sem_b):
    c = lax.axis_index("c")
    s = lax.axis_index("s")
    w = c * NS + s
    nbh = jnp.where(c == CSLOW, NBH_SLOW, NBH_FAST)
    cp_s = pltpu.async_copy(src_hbm.at[w, 0], sidx, sem)
    cp_d = pltpu.async_copy(dst_hbm.at[w, 0], didx, sem)
    pltpu.sync_copy(zeros_hbm, acc.at[pl.ds(s * ROWS_PER_TILE, ROWS_PER_TILE)])
    plsc.subcore_barrier()

    def gather(j, buf):
        pltpu.sync_copy(g_hbm.at[sidx.at[j]], buf)

    def scat(j, buf, sem_g):
        return pltpu.async_copy(buf, acc.at[didx.at[j]], sem_g, add=True)

    def wait_scat(j, buf, sem_g):
        pltpu.make_async_copy(buf, acc.at[didx.at[j]], sem_g).wait()

    for h in range(NH):
        cp_s.wait()
        cp_d.wait()
        gather(0, rows_a)
        scat(0, rows_a, sem_a)

        @pl.loop(0, nbh // 2 - 1)
        def _(p):
            gather(2 * p + 1, rows_b)
            wait_scat(2 * p, rows_a, sem_a)
            scat(2 * p + 1, rows_b, sem_b)
            gather(2 * p + 2, rows_a)
            wait_scat(2 * p + 1, rows_b, sem_b)
            scat(2 * p + 2, rows_a, sem_a)

        gather(nbh - 1, rows_b)
        wait_scat(nbh - 2, rows_a, sem_a)
        scat(nbh - 1, rows_b, sem_b)
        wait_scat(nbh - 1, rows_b, sem_b)
        if h + 1 < NH:
            cp_s = pltpu.async_copy(src_hbm.at[w, h + 1], sidx, sem)
            cp_d = pltpu.async_copy(dst_hbm.at[w, h + 1], didx, sem)

    plsc.subcore_barrier()
    pltpu.sync_copy(acc.at[pl.ds(s * ROWS_PER_TILE, ROWS_PER_TILE)],
                    out_hbm.at[c, pl.ds(s * ROWS_PER_TILE, ROWS_PER_TILE)])


_R = 1024
_NBLK = N_PAD // _R


def _mm_body(x_ref, w_ref, o_ref):
    o_ref[...] = jnp.dot(x_ref[...], w_ref[...],
                         preferred_element_type=jnp.float32)


def _matmul(x, w):
    return pl.pallas_call(
        _mm_body,
        grid=(_NBLK,),
        in_specs=[pl.BlockSpec((_R, D), lambda i: (i, 0)),
                  pl.BlockSpec((D, D), lambda i: (0, 0))],
        out_specs=pl.BlockSpec((_R, D), lambda i: (i, 0)),
        out_shape=jax.ShapeDtypeStruct((N_PAD, D), jnp.float32),
    )(x, w)


def _norm_body(parts_ref, h_ref, g_ref, dinv_ref):
    deg = jnp.sum(parts_ref[...], axis=0) + 1.0
    dinv = lax.rsqrt(deg)[:, None]
    g_ref[...] = h_ref[...] * dinv
    dinv_ref[...] = dinv


def _norm(parts, h):
    return pl.pallas_call(
        _norm_body,
        grid=(_NBLK,),
        in_specs=[pl.BlockSpec((NW, _R), lambda i: (0, i)),
                  pl.BlockSpec((_R, D), lambda i: (i, 0))],
        out_specs=[pl.BlockSpec((_R, D), lambda i: (i, 0)),
                   pl.BlockSpec((_R, 1), lambda i: (i, 0))],
        out_shape=[jax.ShapeDtypeStruct((N_PAD, D), jnp.float32),
                   jax.ShapeDtypeStruct((N_PAD, 1), jnp.float32)],
    )(parts, h)


def _mid_body(h1_ref, acc_ref, dinv_ref, b_ref, w_ref, h2_ref, g2_ref):
    dinv = dinv_ref[...]
    accs = acc_ref[...]
    h1 = jnp.maximum(dinv * (accs[0] + accs[1]) + (dinv * dinv) * h1_ref[...]
                     + b_ref[...], 0.0)
    h2 = jnp.dot(h1, w_ref[...], preferred_element_type=jnp.float32)
    h2_ref[...] = h2
    g2_ref[...] = h2 * dinv


def _mid(h1raw, acc, dinv, b1, w2):
    return pl.pallas_call(
        _mid_body,
        grid=(_NBLK,),
        in_specs=[pl.BlockSpec((_R, D), lambda i: (i, 0)),
                  pl.BlockSpec((NC, _R, D), lambda i: (0, i, 0)),
                  pl.BlockSpec((_R, 1), lambda i: (i, 0)),
                  pl.BlockSpec((1, D), lambda i: (0, 0)),
                  pl.BlockSpec((D, D), lambda i: (0, 0))],
        out_specs=[pl.BlockSpec((_R, D), lambda i: (i, 0)),
                   pl.BlockSpec((_R, D), lambda i: (i, 0))],
        out_shape=[jax.ShapeDtypeStruct((N_PAD, D), jnp.float32),
                   jax.ShapeDtypeStruct((N_PAD, D), jnp.float32)],
    )(h1raw, acc, dinv, b1, w2)


def _final_body(h2_ref, acc_ref, dinv_ref, b_ref, o_ref):
    dinv = dinv_ref[...]
    accs = acc_ref[...]
    o_ref[...] = jnp.maximum(dinv * (accs[0] + accs[1])
                             + (dinv * dinv) * h2_ref[...] + b_ref[...], 0.0)


def _final(h2raw, acc, dinv, b2):
    return pl.pallas_call(
        _final_body,
        grid=(_NBLK,),
        in_specs=[pl.BlockSpec((_R, D), lambda i: (i, 0)),
                  pl.BlockSpec((NC, _R, D), lambda i: (0, i, 0)),
                  pl.BlockSpec((_R, 1), lambda i: (i, 0)),
                  pl.BlockSpec((1, D), lambda i: (0, 0))],
        out_specs=pl.BlockSpec((_R, D), lambda i: (i, 0)),
        out_shape=jax.ShapeDtypeStruct((N_PAD, D), jnp.float32),
    )(h2raw, acc, dinv, b2)


def kernel(x, edge_index, W1, b1, W2, b2):
    ei = edge_index.astype(jnp.int32)
    pad = E_PAD - E_REAL
    src_p = jnp.concatenate([ei[0], jnp.zeros((pad,), jnp.int32)])
    dst_p = jnp.concatenate(
        [ei[1], N_REAL + (jnp.arange(pad, dtype=jnp.int32) % (N_PAD - N_REAL))])
    src_c = src_p.reshape(N_CHUNKS, EB)
    dst_c = dst_p.reshape(N_CHUNKS, EB)

    def pack(chunks):
        slow_total = NS * NB_SLOW
        per_w = []
        for w in range(NW):
            cc, ss = divmod(w, NS)
            if cc == CSLOW:
                blk = chunks[ss * NB_SLOW:(ss + 1) * NB_SLOW]
                blk = blk.reshape(NH, NBH_SLOW, EB)
                fill = jnp.zeros((NH, NBH_FAST - NBH_SLOW, EB), jnp.int32)
                blk = jnp.concatenate([blk, fill], axis=1)
            else:
                base = slow_total + ss * NB_FAST
                blk = chunks[base:base + NB_FAST].reshape(NH, NBH_FAST, EB)
            per_w.append(blk)
        return jnp.stack(per_w)

    src_w = pack(src_c)
    dst_w = pack(dst_c)
    dst_flat = dst_p.reshape(NW, EPW)
    zeros_tile = jnp.zeros((ROWS_PER_TILE, D), jnp.float32)

    x_pad = jnp.concatenate([x, jnp.zeros((N_PAD - N_REAL, D), jnp.float32)])
    b1r = b1.reshape(1, D)
    b2r = b2.reshape(1, D)

    deg_parts = _deg_kernel(dst_flat)
    h1raw = _matmul(x_pad, W1)
    g1, dinv = _norm(deg_parts, h1raw)
    acc1 = _scatter_kernel(g1, src_w, dst_w, zeros_tile)
    h2raw, g2 = _mid(h1raw, acc1, dinv, b1r, W2)
    acc2 = _scatter_kernel(g2, src_w, dst_w, zeros_tile)
    out = _final(h2raw, acc2, dinv, b2r)
    return out[:N_REAL]


# speedup vs baseline: 1.0583x; 1.0583x over previous
import dataclasses
import functools

import jax
import jax.numpy as jnp
from jax import lax
from jax.experimental import pallas as pl
from jax.experimental.pallas import tpu as pltpu
from jax.experimental.pallas import tpu_sc as plsc

N_REAL = 10000
N_PAD = 10240
D = 128
NC, NS = 2, 16
NW = NC * NS
EB = 128
E_REAL = 320000
NH = 2
CSLOW = 0
NBH_SLOW = 18
NBH_FAST = 62
NB_SLOW = NH * NBH_SLOW
NB_FAST = NH * NBH_FAST
N_CHUNKS = NS * (NB_SLOW + NB_FAST)
E_PAD = N_CHUNKS * EB
EPW = E_PAD // NW
ROWS_PER_TILE = N_PAD // NS

_mesh = plsc.VectorSubcoreMesh(core_axis_name="c", subcore_axis_name="s")

_sc_params = pltpu.CompilerParams()
if "needs_layout_passes" in pltpu.CompilerParams.__dataclass_fields__:
    _sc_params = dataclasses.replace(_sc_params, needs_layout_passes=False)


@functools.partial(
    pl.kernel,
    mesh=_mesh,
    out_type=jax.ShapeDtypeStruct((NW, N_PAD), jnp.float32),
    compiler_params=_sc_params,
    scratch_types=[
        pltpu.VMEM((EPW,), jnp.int32),
        pltpu.VMEM((N_PAD,), jnp.float32),
        pltpu.SemaphoreType.DMA,
    ],
)
def _deg_kernel(dst_hbm, out_hbm, idx_v, hist_v, sem):
    c = lax.axis_index("c")
    s = lax.axis_index("s")
    w = c * NS + s
    cp = pltpu.async_copy(dst_hbm.at[w], idx_v, sem)
    z16 = jnp.zeros((16,), jnp.float32)

    @pl.loop(0, N_PAD // 16)
    def _(i):
        hist_v[pl.ds(i * 16, 16)] = z16

    cp.wait()
    ones16 = jnp.ones((16,), jnp.float32)

    @pl.loop(0, EPW // 16)
    def _(t):
        idx = idx_v[pl.ds(t * 16, 16)]
        plsc.addupdate_scatter(hist_v, [idx], ones16)

    pltpu.sync_copy(hist_v, out_hbm.at[w])


@functools.partial(
    pl.kernel,
    mesh=_mesh,
    out_type=jax.ShapeDtypeStruct((NC, N_PAD, D), jnp.float32),
    scratch_types=[
        pltpu.VMEM((NBH_FAST, EB), jnp.int32),
        pltpu.VMEM((NBH_FAST, EB), jnp.int32),
        pltpu.VMEM((EB, D), jnp.float32),
        pltpu.VMEM((EB, D), jnp.float32),
        pltpu.VMEM_SHARED((N_PAD, D), jnp.float32),
        pltpu.SemaphoreType.DMA,
        pltpu.SemaphoreType.DMA,
        pltpu.SemaphoreType.DMA,
    ],
)
def _scatter_kernel(g_hbm, src_hbm, dst_hbm, zeros_hbm, out_hbm,
                    sidx, didx, rows_a, rows_b, acc, sem, sem_a, sem_b):
    c = lax.axis_index("c")
    s = lax.axis_index("s")
    w = c * NS + s
    nbh = jnp.where(c == CSLOW, NBH_SLOW, NBH_FAST)
    cp_s = pltpu.async_copy(src_hbm.at[w, 0], sidx, sem)
    cp_d = pltpu.async_copy(dst_hbm.at[w, 0], didx, sem)
    pltpu.sync_copy(zeros_hbm, acc.at[pl.ds(s * ROWS_PER_TILE, ROWS_PER_TILE)])
    plsc.subcore_barrier()

    def gather(j, buf):
        pltpu.sync_copy(g_hbm.at[sidx.at[j]], buf)

    def scat(j, buf, sem_g):
        return pltpu.async_copy(buf, acc.at[didx.at[j]], sem_g, add=True)

    def wait_scat(j, buf, sem_g):
        pltpu.make_async_copy(buf, acc.at[didx.at[j]], sem_g).wait()

    for h in range(NH):
        cp_s.wait()
        cp_d.wait()
        gather(0, rows_a)
        scat(0, rows_a, sem_a)

        @pl.loop(0, nbh // 2 - 1)
        def _(p):
            gather(2 * p + 1, rows_b)
            wait_scat(2 * p, rows_a, sem_a)
            scat(2 * p + 1, rows_b, sem_b)
            gather(2 * p + 2, rows_a)
            wait_scat(2 * p + 1, rows_b, sem_b)
            scat(2 * p + 2, rows_a, sem_a)

        gather(nbh - 1, rows_b)
        wait_scat(nbh - 2, rows_a, sem_a)
        scat(nbh - 1, rows_b, sem_b)
        wait_scat(nbh - 1, rows_b, sem_b)
        if h + 1 < NH:
            cp_s = pltpu.async_copy(src_hbm.at[w, h + 1], sidx, sem)
            cp_d = pltpu.async_copy(dst_hbm.at[w, h + 1], didx, sem)

    plsc.subcore_barrier()
    pltpu.sync_copy(acc.at[pl.ds(s * ROWS_PER_TILE, ROWS_PER_TILE)],
                    out_hbm.at[c, pl.ds(s * ROWS_PER_TILE, ROWS_PER_TILE)])


_R = 1024
_NBLK = N_PAD // _R


def _mm_body(x_ref, w_ref, o_ref):
    o_ref[...] = jnp.dot(x_ref[...], w_ref[...],
                         preferred_element_type=jnp.float32)


def _matmul(x, w):
    return pl.pallas_call(
        _mm_body,
        grid=(_NBLK,),
        in_specs=[pl.BlockSpec((_R, D), lambda i: (i, 0)),
                  pl.BlockSpec((D, D), lambda i: (0, 0))],
        out_specs=pl.BlockSpec((_R, D), lambda i: (i, 0)),
        out_shape=jax.ShapeDtypeStruct((N_PAD, D), jnp.float32),
    )(x, w)


def _norm_body(parts_ref, h_ref, g_ref, dinv_ref):
    deg = jnp.sum(parts_ref[...], axis=0) + 1.0
    dinv = lax.rsqrt(deg)[:, None]
    g_ref[...] = h_ref[...] * dinv
    dinv_ref[...] = dinv


def _norm(parts, h):
    return pl.pallas_call(
        _norm_body,
        grid=(_NBLK,),
        in_specs=[pl.BlockSpec((NW, _R), lambda i: (0, i)),
                  pl.BlockSpec((_R, D), lambda i: (i, 0))],
        out_specs=[pl.BlockSpec((_R, D), lambda i: (i, 0)),
                   pl.BlockSpec((_R, 1), lambda i: (i, 0))],
        out_shape=[jax.ShapeDtypeStruct((N_PAD, D), jnp.float32),
                   jax.ShapeDtypeStruct((N_PAD, 1), jnp.float32)],
    )(parts, h)


def _mid_body(h1_ref, acc_ref, dinv_ref, b_ref, w_ref, h2_ref, g2_ref):
    dinv = dinv_ref[...]
    accs = acc_ref[...]
    h1 = jnp.maximum(dinv * (accs[0] + accs[1]) + (dinv * dinv) * h1_ref[...]
                     + b_ref[...], 0.0)
    h2 = jnp.dot(h1, w_ref[...], preferred_element_type=jnp.float32)
    h2_ref[...] = h2
    g2_ref[...] = h2 * dinv


def _mid(h1raw, acc, dinv, b1, w2):
    return pl.pallas_call(
        _mid_body,
        grid=(_NBLK,),
        in_specs=[pl.BlockSpec((_R, D), lambda i: (i, 0)),
                  pl.BlockSpec((NC, _R, D), lambda i: (0, i, 0)),
                  pl.BlockSpec((_R, 1), lambda i: (i, 0)),
                  pl.BlockSpec((1, D), lambda i: (0, 0)),
                  pl.BlockSpec((D, D), lambda i: (0, 0))],
        out_specs=[pl.BlockSpec((_R, D), lambda i: (i, 0)),
                   pl.BlockSpec((_R, D), lambda i: (i, 0))],
        out_shape=[jax.ShapeDtypeStruct((N_PAD, D), jnp.float32),
                   jax.ShapeDtypeStruct((N_PAD, D), jnp.float32)],
    )(h1raw, acc, dinv, b1, w2)


def _final_body(h2_ref, acc_ref, dinv_ref, b_ref, o_ref):
    dinv = dinv_ref[...]
    accs = acc_ref[...]
    o_ref[...] = jnp.maximum(dinv * (accs[0] + accs[1])
                             + (dinv * dinv) * h2_ref[...] + b_ref[...], 0.0)


def _final(h2raw, acc, dinv, b2):
    return pl.pallas_call(
        _final_body,
        grid=(_NBLK,),
        in_specs=[pl.BlockSpec((_R, D), lambda i: (i, 0)),
                  pl.BlockSpec((NC, _R, D), lambda i: (0, i, 0)),
                  pl.BlockSpec((_R, 1), lambda i: (i, 0)),
                  pl.BlockSpec((1, D), lambda i: (0, 0))],
        out_specs=pl.BlockSpec((_R, D), lambda i: (i, 0)),
        out_shape=jax.ShapeDtypeStruct((N_PAD, D), jnp.float32),
    )(h2raw, acc, dinv, b2)


def kernel(x, edge_index, W1, b1, W2, b2):
    ei = edge_index.astype(jnp.int32)
    pad = E_PAD - E_REAL
    src_p = jnp.concatenate([ei[0], jnp.zeros((pad,), jnp.int32)])
    dst_p = jnp.concatenate(
        [ei[1], N_REAL + (jnp.arange(pad, dtype=jnp.int32) % (N_PAD - N_REAL))])
    src_c = src_p.reshape(N_CHUNKS, EB)
    dst_c = dst_p.reshape(N_CHUNKS, EB)

    def pack(chunks):
        slow_total = NS * NB_SLOW
        per_w = []
        for w in range(NW):
            cc, ss = divmod(w, NS)
            if cc == CSLOW:
                blk = chunks[ss * NB_SLOW:(ss + 1) * NB_SLOW]
                blk = blk.reshape(NH, NBH_SLOW, EB)
                fill = jnp.zeros((NH, NBH_FAST - NBH_SLOW, EB), jnp.int32)
                blk = jnp.concatenate([blk, fill], axis=1)
            else:
                base = slow_total + ss * NB_FAST
                blk = chunks[base:base + NB_FAST].reshape(NH, NBH_FAST, EB)
            per_w.append(blk)
        return jnp.stack(per_w)

    src_w = pack(src_c)
    dst_w = pack(dst_c)
    dst_flat = dst_p.reshape(NW, EPW)
    zeros_tile = jnp.zeros((ROWS_PER_TILE, D), jnp.float32)

    x_pad = jnp.concatenate([x, jnp.zeros((N_PAD - N_REAL, D), jnp.float32)])
    b1r = b1.reshape(1, D)
    b2r = b2.reshape(1, D)

    deg_parts = _deg_kernel(dst_flat)
    h1raw = _matmul(x_pad, W1)
    g1, dinv = _norm(deg_parts, h1raw)
    acc1 = _scatter_kernel(g1, src_w, dst_w, zeros_tile)
    h2raw, g2 = _mid(h1raw, acc1, dinv, b1r, W2)
    acc2 = _scatter_kernel(g2, src_w, dst_w, zeros_tile)
    out = _final(h2raw, acc2, dinv, b2r)
    return out[:N_REAL]


# speedup vs baseline: 1.1769x; 1.1120x over previous
import dataclasses
import functools

import jax
import jax.numpy as jnp
from jax import lax
from jax.experimental import pallas as pl
from jax.experimental.pallas import tpu as pltpu
from jax.experimental.pallas import tpu_sc as plsc

N_REAL = 10000
N_PAD = 10240
D = 128
NC, NS = 2, 16
NW = NC * NS
EB = 128
E_REAL = 320000
NH = 1
NBH = 80
NB = NH * NBH
N_CHUNKS = NW * NB
E_PAD = N_CHUNKS * EB
EPW = E_PAD // NW
ROWS_PER_TILE = N_PAD // NS

_mesh = plsc.VectorSubcoreMesh(core_axis_name="c", subcore_axis_name="s")

_sc_params = pltpu.CompilerParams()
if "needs_layout_passes" in pltpu.CompilerParams.__dataclass_fields__:
    _sc_params = dataclasses.replace(_sc_params, needs_layout_passes=False)


@functools.partial(
    pl.kernel,
    mesh=_mesh,
    out_type=jax.ShapeDtypeStruct((NW, N_PAD), jnp.float32),
    compiler_params=_sc_params,
    scratch_types=[
        pltpu.VMEM((EPW,), jnp.int32),
        pltpu.VMEM((N_PAD,), jnp.float32),
        pltpu.SemaphoreType.DMA,
    ],
)
def _deg_kernel(dst_hbm, out_hbm, idx_v, hist_v, sem):
    c = lax.axis_index("c")
    s = lax.axis_index("s")
    w = c * NS + s
    cp = pltpu.async_copy(dst_hbm.at[w], idx_v, sem)
    z16 = jnp.zeros((16,), jnp.float32)

    @pl.loop(0, N_PAD // 16)
    def _(i):
        hist_v[pl.ds(i * 16, 16)] = z16

    cp.wait()
    ones16 = jnp.ones((16,), jnp.float32)

    @pl.loop(0, EPW // 16)
    def _(t):
        idx = idx_v[pl.ds(t * 16, 16)]
        plsc.addupdate_scatter(hist_v, [idx], ones16)

    pltpu.sync_copy(hist_v, out_hbm.at[w])


@functools.partial(
    pl.kernel,
    mesh=_mesh,
    out_type=jax.ShapeDtypeStruct((NC, N_PAD, D), jnp.float32),
    scratch_types=[
        pltpu.VMEM((NBH, EB), jnp.int32),
        pltpu.VMEM((NBH, EB), jnp.int32),
        pltpu.VMEM((EB, D), jnp.float32),
        pltpu.VMEM_SHARED((N_PAD, D), jnp.float32),
        pltpu.SemaphoreType.DMA,
    ],
)
def _scatter_kernel(g_hbm, src_hbm, dst_hbm, zeros_hbm, out_hbm,
                    sidx, didx, rows, acc, sem):
    c = lax.axis_index("c")
    s = lax.axis_index("s")
    w = c * NS + s
    cp_s = pltpu.async_copy(src_hbm.at[w, 0], sidx, sem)
    cp_d = pltpu.async_copy(dst_hbm.at[w, 0], didx, sem)
    pltpu.sync_copy(zeros_hbm, acc.at[pl.ds(s * ROWS_PER_TILE, ROWS_PER_TILE)])
    plsc.subcore_barrier()

    for h in range(NH):
        cp_s.wait()
        cp_d.wait()

        @pl.loop(0, NBH)
        def _(j):
            pltpu.sync_copy(g_hbm.at[sidx.at[j]], rows)
            pltpu.sync_copy(rows, acc.at[didx.at[j]], add=True)

        if h + 1 < NH:
            cp_s = pltpu.async_copy(src_hbm.at[w, h + 1], sidx, sem)
            cp_d = pltpu.async_copy(dst_hbm.at[w, h + 1], didx, sem)

    plsc.subcore_barrier()
    pltpu.sync_copy(acc.at[pl.ds(s * ROWS_PER_TILE, ROWS_PER_TILE)],
                    out_hbm.at[c, pl.ds(s * ROWS_PER_TILE, ROWS_PER_TILE)])


_R = 1024
_NBLK = N_PAD // _R


def _mm_body(x_ref, w_ref, o_ref):
    o_ref[...] = jnp.dot(x_ref[...], w_ref[...],
                         preferred_element_type=jnp.float32)


def _matmul(x, w):
    return pl.pallas_call(
        _mm_body,
        grid=(_NBLK,),
        in_specs=[pl.BlockSpec((_R, D), lambda i: (i, 0)),
                  pl.BlockSpec((D, D), lambda i: (0, 0))],
        out_specs=pl.BlockSpec((_R, D), lambda i: (i, 0)),
        out_shape=jax.ShapeDtypeStruct((N_PAD, D), jnp.float32),
    )(x, w)


def _norm_body(parts_ref, h_ref, g_ref, dinv_ref):
    deg = jnp.sum(parts_ref[...], axis=0) + 1.0
    dinv = lax.rsqrt(deg)[:, None]
    g_ref[...] = h_ref[...] * dinv
    dinv_ref[...] = dinv


def _norm(parts, h):
    return pl.pallas_call(
        _norm_body,
        grid=(_NBLK,),
        in_specs=[pl.BlockSpec((NW, _R), lambda i: (0, i)),
                  pl.BlockSpec((_R, D), lambda i: (i, 0))],
        out_specs=[pl.BlockSpec((_R, D), lambda i: (i, 0)),
                   pl.BlockSpec((_R, 1), lambda i: (i, 0))],
        out_shape=[jax.ShapeDtypeStruct((N_PAD, D), jnp.float32),
                   jax.ShapeDtypeStruct((N_PAD, 1), jnp.float32)],
    )(parts, h)


def _mid_body(h1_ref, acc_ref, dinv_ref, b_ref, w_ref, h2_ref, g2_ref):
    dinv = dinv_ref[...]
    accs = acc_ref[...]
    h1 = jnp.maximum(dinv * (accs[0] + accs[1]) + (dinv * dinv) * h1_ref[...]
                     + b_ref[...], 0.0)
    h2 = jnp.dot(h1, w_ref[...], preferred_element_type=jnp.float32)
    h2_ref[...] = h2
    g2_ref[...] = h2 * dinv


def _mid(h1raw, acc, dinv, b1, w2):
    return pl.pallas_call(
        _mid_body,
        grid=(_NBLK,),
        in_specs=[pl.BlockSpec((_R, D), lambda i: (i, 0)),
                  pl.BlockSpec((NC, _R, D), lambda i: (0, i, 0)),
                  pl.BlockSpec((_R, 1), lambda i: (i, 0)),
                  pl.BlockSpec((1, D), lambda i: (0, 0)),
                  pl.BlockSpec((D, D), lambda i: (0, 0))],
        out_specs=[pl.BlockSpec((_R, D), lambda i: (i, 0)),
                   pl.BlockSpec((_R, D), lambda i: (i, 0))],
        out_shape=[jax.ShapeDtypeStruct((N_PAD, D), jnp.float32),
                   jax.ShapeDtypeStruct((N_PAD, D), jnp.float32)],
    )(h1raw, acc, dinv, b1, w2)


def _final_body(h2_ref, acc_ref, dinv_ref, b_ref, o_ref):
    dinv = dinv_ref[...]
    accs = acc_ref[...]
    o_ref[...] = jnp.maximum(dinv * (accs[0] + accs[1])
                             + (dinv * dinv) * h2_ref[...] + b_ref[...], 0.0)


def _final(h2raw, acc, dinv, b2):
    return pl.pallas_call(
        _final_body,
        grid=(_NBLK,),
        in_specs=[pl.BlockSpec((_R, D), lambda i: (i, 0)),
                  pl.BlockSpec((NC, _R, D), lambda i: (0, i, 0)),
                  pl.BlockSpec((_R, 1), lambda i: (i, 0)),
                  pl.BlockSpec((1, D), lambda i: (0, 0))],
        out_specs=pl.BlockSpec((_R, D), lambda i: (i, 0)),
        out_shape=jax.ShapeDtypeStruct((N_PAD, D), jnp.float32),
    )(h2raw, acc, dinv, b2)


def kernel(x, edge_index, W1, b1, W2, b2):
    ei = edge_index.astype(jnp.int32)
    pad = E_PAD - E_REAL
    src_p = jnp.concatenate([ei[0], jnp.zeros((pad,), jnp.int32)])
    dst_p = jnp.concatenate(
        [ei[1], N_REAL + (jnp.arange(pad, dtype=jnp.int32) % (N_PAD - N_REAL))])
    src_w = src_p.reshape(NW, NH, NBH, EB)
    dst_w = dst_p.reshape(NW, NH, NBH, EB)
    dst_flat = dst_p.reshape(NW, EPW)
    zeros_tile = jnp.zeros((ROWS_PER_TILE, D), jnp.float32)

    x_pad = jnp.concatenate([x, jnp.zeros((N_PAD - N_REAL, D), jnp.float32)])
    b1r = b1.reshape(1, D)
    b2r = b2.reshape(1, D)

    deg_parts = _deg_kernel(dst_flat)
    h1raw = _matmul(x_pad, W1)
    g1, dinv = _norm(deg_parts, h1raw)
    acc1 = _scatter_kernel(g1, src_w, dst_w, zeros_tile)
    h2raw, g2 = _mid(h1raw, acc1, dinv, b1r, W2)
    acc2 = _scatter_kernel(g2, src_w, dst_w, zeros_tile)
    out = _final(h2raw, acc2, dinv, b2r)
    return out[:N_REAL]


# speedup vs baseline: 1.6297x; 1.3848x over previous
import dataclasses
import functools

import jax
import jax.numpy as jnp
from jax import lax
from jax.experimental import pallas as pl
from jax.experimental.pallas import tpu as pltpu
from jax.experimental.pallas import tpu_sc as plsc

N_REAL = 10000
N_PAD = 10240
D = 128
NC, NS = 2, 16
NW = NC * NS
EB = 128
E_REAL = 320000
NH = 1
NBH = 79
NB = NH * NBH
N_CHUNKS = NW * NB
E_PAD = N_CHUNKS * EB
EPW = E_PAD // NW
ROWS_PER_TILE = N_PAD // NS

_mesh = plsc.VectorSubcoreMesh(core_axis_name="c", subcore_axis_name="s")

_sc_params = pltpu.CompilerParams()
if "needs_layout_passes" in pltpu.CompilerParams.__dataclass_fields__:
    _sc_params = dataclasses.replace(_sc_params, needs_layout_passes=False)


@functools.partial(
    pl.kernel,
    mesh=_mesh,
    out_type=jax.ShapeDtypeStruct((NW, N_PAD), jnp.float32),
    compiler_params=_sc_params,
    scratch_types=[
        pltpu.VMEM((EPW,), jnp.int32),
        pltpu.VMEM((N_PAD,), jnp.float32),
        pltpu.SemaphoreType.DMA,
    ],
)
def _deg_kernel(dst_hbm, out_hbm, idx_v, hist_v, sem):
    c = lax.axis_index("c")
    s = lax.axis_index("s")
    w = c * NS + s
    cp = pltpu.async_copy(dst_hbm.at[w], idx_v, sem)
    z16 = jnp.zeros((16,), jnp.float32)

    @pl.loop(0, N_PAD // 16)
    def _(i):
        hist_v[pl.ds(i * 16, 16)] = z16

    cp.wait()
    ones16 = jnp.ones((16,), jnp.float32)

    @pl.loop(0, EPW // 16)
    def _(t):
        idx = idx_v[pl.ds(t * 16, 16)]
        plsc.addupdate_scatter(hist_v, [idx], ones16)

    pltpu.sync_copy(hist_v, out_hbm.at[w])


@functools.partial(
    pl.kernel,
    mesh=_mesh,
    out_type=jax.ShapeDtypeStruct((NC, N_PAD, D), jnp.float32),
    scratch_types=[
        pltpu.VMEM((NBH, EB), jnp.int32),
        pltpu.VMEM((NBH, EB), jnp.int32),
        pltpu.VMEM((EB, D), jnp.float32),
        pltpu.VMEM_SHARED((N_PAD, D), jnp.float32),
        pltpu.SemaphoreType.DMA,
    ],
)
def _scatter_kernel(g_hbm, src_hbm, dst_hbm, zeros_hbm, out_hbm,
                    sidx, didx, rows, acc, sem):
    c = lax.axis_index("c")
    s = lax.axis_index("s")
    w = c * NS + s
    cp_s = pltpu.async_copy(src_hbm.at[w, 0], sidx, sem)
    cp_d = pltpu.async_copy(dst_hbm.at[w, 0], didx, sem)
    pltpu.sync_copy(zeros_hbm, acc.at[pl.ds(s * ROWS_PER_TILE, ROWS_PER_TILE)])
    plsc.subcore_barrier()

    for h in range(NH):
        cp_s.wait()
        cp_d.wait()

        @pl.loop(0, NBH)
        def _(j):
            pltpu.sync_copy(g_hbm.at[sidx.at[j]], rows)
            pltpu.sync_copy(rows, acc.at[didx.at[j]], add=True)

        if h + 1 < NH:
            cp_s = pltpu.async_copy(src_hbm.at[w, h + 1], sidx, sem)
            cp_d = pltpu.async_copy(dst_hbm.at[w, h + 1], didx, sem)

    plsc.subcore_barrier()
    pltpu.sync_copy(acc.at[pl.ds(s * ROWS_PER_TILE, ROWS_PER_TILE)],
                    out_hbm.at[c, pl.ds(s * ROWS_PER_TILE, ROWS_PER_TILE)])


_R = 1024
_NBLK = N_PAD // _R


def _mm_body(x_ref, w_ref, o_ref):
    o_ref[...] = jnp.dot(x_ref[...], w_ref[...],
                         preferred_element_type=jnp.float32)


def _matmul(x, w):
    return pl.pallas_call(
        _mm_body,
        grid=(_NBLK,),
        in_specs=[pl.BlockSpec((_R, D), lambda i: (i, 0)),
                  pl.BlockSpec((D, D), lambda i: (0, 0))],
        out_specs=pl.BlockSpec((_R, D), lambda i: (i, 0)),
        out_shape=jax.ShapeDtypeStruct((N_PAD, D), jnp.float32),
    )(x, w)


def _norm_body(parts_ref, h_ref, g_ref, dinv_ref):
    deg = jnp.sum(parts_ref[...], axis=0) + 1.0
    dinv = lax.rsqrt(deg)[:, None]
    g_ref[...] = h_ref[...] * dinv
    dinv_ref[...] = dinv


def _norm(parts, h):
    return pl.pallas_call(
        _norm_body,
        grid=(_NBLK,),
        in_specs=[pl.BlockSpec((NW, _R), lambda i: (0, i)),
                  pl.BlockSpec((_R, D), lambda i: (i, 0))],
        out_specs=[pl.BlockSpec((_R, D), lambda i: (i, 0)),
                   pl.BlockSpec((_R, 1), lambda i: (i, 0))],
        out_shape=[jax.ShapeDtypeStruct((N_PAD, D), jnp.float32),
                   jax.ShapeDtypeStruct((N_PAD, 1), jnp.float32)],
    )(parts, h)


def _mid_body(h1_ref, acc_ref, dinv_ref, b_ref, w_ref, h2_ref, g2_ref):
    dinv = dinv_ref[...]
    accs = acc_ref[...]
    h1 = jnp.maximum(dinv * (accs[0] + accs[1]) + (dinv * dinv) * h1_ref[...]
                     + b_ref[...], 0.0)
    h2 = jnp.dot(h1, w_ref[...], preferred_element_type=jnp.float32)
    h2_ref[...] = h2
    g2_ref[...] = h2 * dinv


def _mid(h1raw, acc, dinv, b1, w2):
    return pl.pallas_call(
        _mid_body,
        grid=(_NBLK,),
        in_specs=[pl.BlockSpec((_R, D), lambda i: (i, 0)),
                  pl.BlockSpec((NC, _R, D), lambda i: (0, i, 0)),
                  pl.BlockSpec((_R, 1), lambda i: (i, 0)),
                  pl.BlockSpec((1, D), lambda i: (0, 0)),
                  pl.BlockSpec((D, D), lambda i: (0, 0))],
        out_specs=[pl.BlockSpec((_R, D), lambda i: (i, 0)),
                   pl.BlockSpec((_R, D), lambda i: (i, 0))],
        out_shape=[jax.ShapeDtypeStruct((N_PAD, D), jnp.float32),
                   jax.ShapeDtypeStruct((N_PAD, D), jnp.float32)],
    )(h1raw, acc, dinv, b1, w2)


def _final_body(h2_ref, acc_ref, dinv_ref, b_ref, o_ref):
    dinv = dinv_ref[...]
    accs = acc_ref[...]
    o_ref[...] = jnp.maximum(dinv * (accs[0] + accs[1])
                             + (dinv * dinv) * h2_ref[...] + b_ref[...], 0.0)


def _final(h2raw, acc, dinv, b2):
    return pl.pallas_call(
        _final_body,
        grid=(_NBLK,),
        in_specs=[pl.BlockSpec((_R, D), lambda i: (i, 0)),
                  pl.BlockSpec((NC, _R, D), lambda i: (0, i, 0)),
                  pl.BlockSpec((_R, 1), lambda i: (i, 0)),
                  pl.BlockSpec((1, D), lambda i: (0, 0))],
        out_specs=pl.BlockSpec((_R, D), lambda i: (i, 0)),
        out_shape=jax.ShapeDtypeStruct((N_PAD, D), jnp.float32),
    )(h2raw, acc, dinv, b2)


def kernel(x, edge_index, W1, b1, W2, b2):
    ei = edge_index.astype(jnp.int32)
    pad = E_PAD - E_REAL
    src_p = jnp.concatenate([ei[0], jnp.zeros((pad,), jnp.int32)])
    dst_p = jnp.concatenate(
        [ei[1], N_REAL + (jnp.arange(pad, dtype=jnp.int32) % (N_PAD - N_REAL))])
    src_w = src_p.reshape(NW, NH, NBH, EB)
    dst_w = dst_p.reshape(NW, NH, NBH, EB)
    dst_flat = dst_p.reshape(NW, EPW)
    zeros_tile = jnp.zeros((ROWS_PER_TILE, D), jnp.float32)

    x_pad = jnp.concatenate([x, jnp.zeros((N_PAD - N_REAL, D), jnp.float32)])
    b1r = b1.reshape(1, D)
    b2r = b2.reshape(1, D)

    deg_parts = _deg_kernel(dst_flat)
    h1raw = _matmul(x_pad, W1)
    g1, dinv = _norm(deg_parts, h1raw)
    acc1 = _scatter_kernel(g1, src_w, dst_w, zeros_tile)
    h2raw, g2 = _mid(h1raw, acc1, dinv, b1r, W2)
    acc2 = _scatter_kernel(g2, src_w, dst_w, zeros_tile)
    out = _final(h2raw, acc2, dinv, b2r)
    return out[:N_REAL]


# speedup vs baseline: 3.8341x; 2.3526x over previous
import dataclasses
import functools

import jax
import jax.numpy as jnp
from jax import lax
from jax.experimental import pallas as pl
from jax.experimental.pallas import tpu as pltpu
from jax.experimental.pallas import tpu_sc as plsc

N_REAL = 10000
N_PAD = 10240
D = 128
NC, NS = 2, 16
NW = NC * NS
EB = 128
E_REAL = 320000
NH = 2
NBH = 40
NB = NH * NBH
N_CHUNKS = NW * NB
E_PAD = N_CHUNKS * EB
EPW = E_PAD // NW
ROWS_PER_TILE = N_PAD // NS

_mesh = plsc.VectorSubcoreMesh(core_axis_name="c", subcore_axis_name="s")

_sc_params = pltpu.CompilerParams()
if "needs_layout_passes" in pltpu.CompilerParams.__dataclass_fields__:
    _sc_params = dataclasses.replace(_sc_params, needs_layout_passes=False)


@functools.partial(
    pl.kernel,
    mesh=_mesh,
    out_type=jax.ShapeDtypeStruct((NW, N_PAD), jnp.float32),
    compiler_params=_sc_params,
    scratch_types=[
        pltpu.VMEM((EPW,), jnp.int32),
        pltpu.VMEM((N_PAD,), jnp.float32),
        pltpu.SemaphoreType.DMA,
    ],
)
def _deg_kernel(dst_hbm, out_hbm, idx_v, hist_v, sem):
    c = lax.axis_index("c")
    s = lax.axis_index("s")
    w = c * NS + s
    cp = pltpu.async_copy(dst_hbm.at[w], idx_v, sem)
    z16 = jnp.zeros((16,), jnp.float32)

    @pl.loop(0, N_PAD // 16)
    def _(i):
        hist_v[pl.ds(i * 16, 16)] = z16

    cp.wait()
    ones16 = jnp.ones((16,), jnp.float32)

    @pl.loop(0, EPW // 16)
    def _(t):
        idx = idx_v[pl.ds(t * 16, 16)]
        plsc.addupdate_scatter(hist_v, [idx], ones16)

    pltpu.sync_copy(hist_v, out_hbm.at[w])


@functools.partial(
    pl.kernel,
    mesh=_mesh,
    out_type=jax.ShapeDtypeStruct((NC, N_PAD, D), jnp.float32),
    scratch_types=[
        pltpu.VMEM((NBH, EB), jnp.int32),
        pltpu.VMEM((NBH, EB), jnp.int32),
        pltpu.VMEM((EB, D), jnp.float32),
        pltpu.VMEM((EB, D), jnp.float32),
        pltpu.VMEM_SHARED((N_PAD, D), jnp.float32),
        pltpu.SemaphoreType.DMA,
        pltpu.SemaphoreType.DMA,
        pltpu.SemaphoreType.DMA,
    ],
)
def _scatter_kernel(g_hbm, src_hbm, dst_hbm, zeros_hbm, out_hbm,
                    sidx, didx, rows_a, rows_b, acc, sem, sem_a, sem_b):
    c = lax.axis_index("c")
    s = lax.axis_index("s")
    w = c * NS + s
    cp_s = pltpu.async_copy(src_hbm.at[w, 0], sidx, sem)
    cp_d = pltpu.async_copy(dst_hbm.at[w, 0], didx, sem)
    pltpu.sync_copy(zeros_hbm, acc.at[pl.ds(s * ROWS_PER_TILE, ROWS_PER_TILE)])
    plsc.subcore_barrier()

    def gather(j, buf):
        pltpu.sync_copy(g_hbm.at[sidx.at[j]], buf)

    def scat(j, buf, sem_g):
        pltpu.async_copy(buf, acc.at[didx.at[j]], sem_g, add=True)

    def wait_scat(j, buf, sem_g):
        pltpu.make_async_copy(buf, acc.at[didx.at[j]], sem_g).wait()

    for h in range(NH):
        cp_s.wait()
        cp_d.wait()
        gather(0, rows_a)
        scat(0, rows_a, sem_a)

        @pl.loop(0, NBH // 2 - 1)
        def _(p):
            gather(2 * p + 1, rows_b)
            wait_scat(2 * p, rows_a, sem_a)
            scat(2 * p + 1, rows_b, sem_b)
            gather(2 * p + 2, rows_a)
            wait_scat(2 * p + 1, rows_b, sem_b)
            scat(2 * p + 2, rows_a, sem_a)

        gather(NBH - 1, rows_b)
        wait_scat(NBH - 2, rows_a, sem_a)
        scat(NBH - 1, rows_b, sem_b)
        wait_scat(NBH - 1, rows_b, sem_b)
        if h + 1 < NH:
            cp_s = pltpu.async_copy(src_hbm.at[w, h + 1], sidx, sem)
            cp_d = pltpu.async_copy(dst_hbm.at[w, h + 1], didx, sem)

    plsc.subcore_barrier()
    pltpu.sync_copy(acc.at[pl.ds(s * ROWS_PER_TILE, ROWS_PER_TILE)],
                    out_hbm.at[c, pl.ds(s * ROWS_PER_TILE, ROWS_PER_TILE)])


_R = 1024
_NBLK = N_PAD // _R


def _mm_body(x_ref, w_ref, o_ref):
    o_ref[...] = jnp.dot(x_ref[...], w_ref[...],
                         preferred_element_type=jnp.float32)


def _matmul(x, w):
    return pl.pallas_call(
        _mm_body,
        grid=(_NBLK,),
        in_specs=[pl.BlockSpec((_R, D), lambda i: (i, 0)),
                  pl.BlockSpec((D, D), lambda i: (0, 0))],
        out_specs=pl.BlockSpec((_R, D), lambda i: (i, 0)),
        out_shape=jax.ShapeDtypeStruct((N_PAD, D), jnp.float32),
    )(x, w)


def _norm_body(parts_ref, h_ref, g_ref, dinv_ref):
    deg = jnp.sum(parts_ref[...], axis=0) + 1.0
    dinv = lax.rsqrt(deg)[:, None]
    g_ref[...] = h_ref[...] * dinv
    dinv_ref[...] = dinv


def _norm(parts, h):
    return pl.pallas_call(
        _norm_body,
        grid=(_NBLK,),
        in_specs=[pl.BlockSpec((NW, _R), lambda i: (0, i)),
                  pl.BlockSpec((_R, D), lambda i: (i, 0))],
        out_specs=[pl.BlockSpec((_R, D), lambda i: (i, 0)),
                   pl.BlockSpec((_R, 1), lambda i: (i, 0))],
        out_shape=[jax.ShapeDtypeStruct((N_PAD, D), jnp.float32),
                   jax.ShapeDtypeStruct((N_PAD, 1), jnp.float32)],
    )(parts, h)


def _mid_body(h1_ref, acc_ref, dinv_ref, b_ref, w_ref, h2_ref, g2_ref):
    dinv = dinv_ref[...]
    accs = acc_ref[...]
    h1 = jnp.maximum(dinv * (accs[0] + accs[1]) + (dinv * dinv) * h1_ref[...]
                     + b_ref[...], 0.0)
    h2 = jnp.dot(h1, w_ref[...], preferred_element_type=jnp.float32)
    h2_ref[...] = h2
    g2_ref[...] = h2 * dinv


def _mid(h1raw, acc, dinv, b1, w2):
    return pl.pallas_call(
        _mid_body,
        grid=(_NBLK,),
        in_specs=[pl.BlockSpec((_R, D), lambda i: (i, 0)),
                  pl.BlockSpec((NC, _R, D), lambda i: (0, i, 0)),
                  pl.BlockSpec((_R, 1), lambda i: (i, 0)),
                  pl.BlockSpec((1, D), lambda i: (0, 0)),
                  pl.BlockSpec((D, D), lambda i: (0, 0))],
        out_specs=[pl.BlockSpec((_R, D), lambda i: (i, 0)),
                   pl.BlockSpec((_R, D), lambda i: (i, 0))],
        out_shape=[jax.ShapeDtypeStruct((N_PAD, D), jnp.float32),
                   jax.ShapeDtypeStruct((N_PAD, D), jnp.float32)],
    )(h1raw, acc, dinv, b1, w2)


def _final_body(h2_ref, acc_ref, dinv_ref, b_ref, o_ref):
    dinv = dinv_ref[...]
    accs = acc_ref[...]
    o_ref[...] = jnp.maximum(dinv * (accs[0] + accs[1])
                             + (dinv * dinv) * h2_ref[...] + b_ref[...], 0.0)


def _final(h2raw, acc, dinv, b2):
    return pl.pallas_call(
        _final_body,
        grid=(_NBLK,),
        in_specs=[pl.BlockSpec((_R, D), lambda i: (i, 0)),
                  pl.BlockSpec((NC, _R, D), lambda i: (0, i, 0)),
                  pl.BlockSpec((_R, 1), lambda i: (i, 0)),
                  pl.BlockSpec((1, D), lambda i: (0, 0))],
        out_specs=pl.BlockSpec((_R, D), lambda i: (i, 0)),
        out_shape=jax.ShapeDtypeStruct((N_PAD, D), jnp.float32),
    )(h2raw, acc, dinv, b2)


def kernel(x, edge_index, W1, b1, W2, b2):
    ei = edge_index.astype(jnp.int32)
    pad = E_PAD - E_REAL
    src_p = jnp.concatenate(
        [ei[0], jnp.arange(pad, dtype=jnp.int32) % N_REAL])
    dst_p = jnp.concatenate(
        [ei[1], N_REAL + (jnp.arange(pad, dtype=jnp.int32) % (N_PAD - N_REAL))])
    src_w = src_p.reshape(NW, NH, NBH, EB)
    dst_w = dst_p.reshape(NW, NH, NBH, EB)
    dst_flat = dst_p.reshape(NW, EPW)
    zeros_tile = jnp.zeros((ROWS_PER_TILE, D), jnp.float32)

    x_pad = jnp.concatenate([x, jnp.zeros((N_PAD - N_REAL, D), jnp.float32)])
    b1r = b1.reshape(1, D)
    b2r = b2.reshape(1, D)

    deg_parts = _deg_kernel(dst_flat)
    h1raw = _matmul(x_pad, W1)
    g1, dinv = _norm(deg_parts, h1raw)
    acc1 = _scatter_kernel(g1, src_w, dst_w, zeros_tile)
    h2raw, g2 = _mid(h1raw, acc1, dinv, b1r, W2)
    acc2 = _scatter_kernel(g2, src_w, dst_w, zeros_tile)
    out = _final(h2raw, acc2, dinv, b2r)
    return out[:N_REAL]


# speedup vs baseline: 3.8982x; 1.0167x over previous
import dataclasses
import functools

import jax
import jax.numpy as jnp
from jax import lax
from jax.experimental import pallas as pl
from jax.experimental.pallas import tpu as pltpu
from jax.experimental.pallas import tpu_sc as plsc

N_REAL = 10000
N_PAD = 10240
D = 128
NC, NS = 2, 16
NW = NC * NS
EB = 128
E_REAL = 320000
NH = 2
NBH = 40
NB = NH * NBH
N_CHUNKS = NW * NB
E_PAD = N_CHUNKS * EB
EPW = E_PAD // NW
ROWS_PER_TILE = N_PAD // NS

_mesh = plsc.VectorSubcoreMesh(core_axis_name="c", subcore_axis_name="s")

_sc_params = pltpu.CompilerParams()
if "needs_layout_passes" in pltpu.CompilerParams.__dataclass_fields__:
    _sc_params = dataclasses.replace(_sc_params, needs_layout_passes=False)


@functools.partial(
    pl.kernel,
    mesh=_mesh,
    out_type=jax.ShapeDtypeStruct((NW, N_PAD), jnp.float32),
    compiler_params=_sc_params,
    scratch_types=[
        pltpu.VMEM((EPW,), jnp.int32),
        pltpu.VMEM((N_PAD,), jnp.float32),
        pltpu.SemaphoreType.DMA,
    ],
)
def _deg_kernel(dst_hbm, out_hbm, idx_v, hist_v, sem):
    c = lax.axis_index("c")
    s = lax.axis_index("s")
    w = c * NS + s
    cp = pltpu.async_copy(dst_hbm.at[w], idx_v, sem)
    z16 = jnp.zeros((16,), jnp.float32)

    @pl.loop(0, N_PAD // 16)
    def _(i):
        hist_v[pl.ds(i * 16, 16)] = z16

    cp.wait()
    ones16 = jnp.ones((16,), jnp.float32)

    @pl.loop(0, EPW // 16)
    def _(t):
        idx = idx_v[pl.ds(t * 16, 16)]
        plsc.addupdate_scatter(hist_v, [idx], ones16)

    pltpu.sync_copy(hist_v, out_hbm.at[w])


@functools.partial(
    pl.kernel,
    mesh=_mesh,
    out_type=jax.ShapeDtypeStruct((NC, N_PAD, D), jnp.float32),
    scratch_types=[
        pltpu.VMEM((NBH, EB), jnp.int32),
        pltpu.VMEM((NBH, EB), jnp.int32),
        pltpu.VMEM((EB, D), jnp.float32),
        pltpu.VMEM((EB, D), jnp.float32),
        pltpu.VMEM_SHARED((N_PAD, D), jnp.float32),
        pltpu.SemaphoreType.DMA,
        pltpu.SemaphoreType.DMA,
        pltpu.SemaphoreType.DMA,
    ],
)
def _scatter_kernel(g_hbm, src_hbm, dst_hbm, zeros_hbm, out_hbm,
                    sidx, didx, rows_a, rows_b, acc, sem, sem_a, sem_b):
    c = lax.axis_index("c")
    s = lax.axis_index("s")
    w = c * NS + s
    cp_s = pltpu.async_copy(src_hbm.at[w, 0], sidx, sem)
    cp_d = pltpu.async_copy(dst_hbm.at[w, 0], didx, sem)
    pltpu.sync_copy(zeros_hbm, acc.at[pl.ds(s * ROWS_PER_TILE, ROWS_PER_TILE)])
    plsc.subcore_barrier()

    def gather(j, buf):
        pltpu.sync_copy(g_hbm.at[sidx.at[j]], buf)

    def scat(j, buf, sem_g):
        pltpu.async_copy(buf, acc.at[didx.at[j]], sem_g, add=True)

    def wait_scat(j, buf, sem_g):
        pltpu.make_async_copy(buf, acc.at[didx.at[j]], sem_g).wait()

    for h in range(NH):
        cp_s.wait()
        cp_d.wait()
        gather(0, rows_a)
        scat(0, rows_a, sem_a)

        @pl.loop(0, NBH // 2 - 1)
        def _(p):
            gather(2 * p + 1, rows_b)
            wait_scat(2 * p, rows_a, sem_a)
            scat(2 * p + 1, rows_b, sem_b)
            gather(2 * p + 2, rows_a)
            wait_scat(2 * p + 1, rows_b, sem_b)
            scat(2 * p + 2, rows_a, sem_a)

        gather(NBH - 1, rows_b)
        wait_scat(NBH - 2, rows_a, sem_a)
        scat(NBH - 1, rows_b, sem_b)
        wait_scat(NBH - 1, rows_b, sem_b)
        if h + 1 < NH:
            cp_s = pltpu.async_copy(src_hbm.at[w, h + 1], sidx, sem)
            cp_d = pltpu.async_copy(dst_hbm.at[w, h + 1], didx, sem)

    plsc.subcore_barrier()
    pltpu.sync_copy(acc.at[pl.ds(s * ROWS_PER_TILE, ROWS_PER_TILE)],
                    out_hbm.at[c, pl.ds(s * ROWS_PER_TILE, ROWS_PER_TILE)])


_R = 2000
_NBLK = N_REAL // _R


def _first_body(x_ref, w_ref, parts_ref, h_ref, g_ref, dinv_ref):
    h = jnp.dot(x_ref[...], w_ref[...], preferred_element_type=jnp.float32)
    deg = jnp.sum(parts_ref[...], axis=1) + 1.0
    dinv = lax.rsqrt(deg)[:, None]
    h_ref[...] = h
    g_ref[...] = h * dinv
    dinv_ref[...] = dinv


def _first(x, w, parts):
    return pl.pallas_call(
        _first_body,
        grid=(_NBLK,),
        in_specs=[pl.BlockSpec((_R, D), lambda i: (i, 0)),
                  pl.BlockSpec((D, D), lambda i: (0, 0)),
                  pl.BlockSpec((_R, NW), lambda i: (i, 0))],
        out_specs=[pl.BlockSpec((_R, D), lambda i: (i, 0)),
                   pl.BlockSpec((_R, D), lambda i: (i, 0)),
                   pl.BlockSpec((_R, 1), lambda i: (i, 0))],
        out_shape=[jax.ShapeDtypeStruct((N_REAL, D), jnp.float32),
                   jax.ShapeDtypeStruct((N_REAL, D), jnp.float32),
                   jax.ShapeDtypeStruct((N_REAL, 1), jnp.float32)],
    )(x, w, parts)


def _mid_body(h1_ref, acc_ref, dinv_ref, b_ref, w_ref, h2_ref, g2_ref):
    dinv = dinv_ref[...]
    accs = acc_ref[...]
    h1 = jnp.maximum(dinv * (accs[0] + accs[1]) + (dinv * dinv) * h1_ref[...]
                     + b_ref[...], 0.0)
    h2 = jnp.dot(h1, w_ref[...], preferred_element_type=jnp.float32)
    h2_ref[...] = h2
    g2_ref[...] = h2 * dinv


def _mid(h1raw, acc, dinv, b1, w2):
    return pl.pallas_call(
        _mid_body,
        grid=(_NBLK,),
        in_specs=[pl.BlockSpec((_R, D), lambda i: (i, 0)),
                  pl.BlockSpec((NC, _R, D), lambda i: (0, i, 0)),
                  pl.BlockSpec((_R, 1), lambda i: (i, 0)),
                  pl.BlockSpec((1, D), lambda i: (0, 0)),
                  pl.BlockSpec((D, D), lambda i: (0, 0))],
        out_specs=[pl.BlockSpec((_R, D), lambda i: (i, 0)),
                   pl.BlockSpec((_R, D), lambda i: (i, 0))],
        out_shape=[jax.ShapeDtypeStruct((N_REAL, D), jnp.float32),
                   jax.ShapeDtypeStruct((N_REAL, D), jnp.float32)],
    )(h1raw, acc, dinv, b1, w2)


def _final_body(h2_ref, acc_ref, dinv_ref, b_ref, o_ref):
    dinv = dinv_ref[...]
    accs = acc_ref[...]
    o_ref[...] = jnp.maximum(dinv * (accs[0] + accs[1])
                             + (dinv * dinv) * h2_ref[...] + b_ref[...], 0.0)


def _final(h2raw, acc, dinv, b2):
    return pl.pallas_call(
        _final_body,
        grid=(_NBLK,),
        in_specs=[pl.BlockSpec((_R, D), lambda i: (i, 0)),
                  pl.BlockSpec((NC, _R, D), lambda i: (0, i, 0)),
                  pl.BlockSpec((_R, 1), lambda i: (i, 0)),
                  pl.BlockSpec((1, D), lambda i: (0, 0))],
        out_specs=pl.BlockSpec((_R, D), lambda i: (i, 0)),
        out_shape=jax.ShapeDtypeStruct((N_REAL, D), jnp.float32),
    )(h2raw, acc, dinv, b2)


def kernel(x, edge_index, W1, b1, W2, b2):
    ei = edge_index.astype(jnp.int32)
    pad = E_PAD - E_REAL
    src_p = jnp.concatenate(
        [ei[0], jnp.arange(pad, dtype=jnp.int32) % N_REAL])
    dst_p = jnp.concatenate(
        [ei[1], N_REAL + (jnp.arange(pad, dtype=jnp.int32) % (N_PAD - N_REAL))])
    src_w = src_p.reshape(NW, NH, NBH, EB)
    dst_w = dst_p.reshape(NW, NH, NBH, EB)
    dst_flat = dst_p.reshape(NW, EPW)
    zeros_tile = jnp.zeros((ROWS_PER_TILE, D), jnp.float32)

    b1r = b1.reshape(1, D)
    b2r = b2.reshape(1, D)

    deg_parts = _deg_kernel(dst_flat)
    h1raw, g1, dinv = _first(x, W1, deg_parts.T)
    acc1 = _scatter_kernel(g1, src_w, dst_w, zeros_tile)
    h2raw, g2 = _mid(h1raw, acc1, dinv, b1r, W2)
    acc2 = _scatter_kernel(g2, src_w, dst_w, zeros_tile)
    return _final(h2raw, acc2, dinv, b2r)
